# Initial kernel scaffold; baseline (speedup 1.0000x reference)
import jax
import jax.numpy as jnp
from jax.experimental import pallas as pl


def kernel(triplets, s_hist_ent, s_hist_len, o_hist_ent, o_hist_len, ent_embeds, rel_embeds, Wih_s, Whh_s, bih_s, bhh_s, Wih_o, Whh_o, bih_o, bhh_o, W_sub, b_sub, W_ob, b_ob):
    raise NotImplementedError("write your pallas kernel here")


# speedup vs baseline: 3.4936x; 3.4936x over previous
import functools
import jax
import jax.numpy as jnp
from jax import lax
from jax.experimental import pallas as pl
from jax.experimental.pallas import tpu as pltpu
from jax.experimental.pallas import tpu_sc as plsc

V = 10000
H = 200
HP = 208
T = 10
K = 64
B = 1024
NC, NS = 2, 16
NW = NC * NS
PAIRS = B * T
PPW = PAIRS // NW
CH = 16
RPW = B // NW
NV = HP // 16

VB = 1024
VP = 10240


def _sc_gather():
    mesh = plsc.VectorSubcoreMesh(core_axis_name="c", subcore_axis_name="s")
    out_type = (
        jax.ShapeDtypeStruct((PAIRS, HP), jnp.float32),
        jax.ShapeDtypeStruct((PAIRS, HP), jnp.float32),
        jax.ShapeDtypeStruct((B, HP), jnp.float32),
        jax.ShapeDtypeStruct((B, HP), jnp.float32),
        jax.ShapeDtypeStruct((B, HP), jnp.float32),
        jax.ShapeDtypeStruct((B, HP), jnp.float32),
    )
    scratch = [
        pltpu.VMEM((CH, K), jnp.int32),
        pltpu.VMEM((K, HP), jnp.float32),
        pltpu.VMEM((CH, HP), jnp.float32),
        pltpu.VMEM((RPW,), jnp.int32),
        pltpu.VMEM((RPW, HP), jnp.float32),
        pltpu.SemaphoreType.DMA,
    ]

    @functools.partial(
        pl.kernel, mesh=mesh, out_type=out_type, scratch_types=scratch,
        compiler_params=pltpu.CompilerParams(use_tc_tiling_on_sc=False))
    def k(he_s, he_o, es_i, rs_i, eo_i, ro_i, ent_t, rel_t,
          neigh_s, neigh_o, es_o, rs_o, eo_o, ro_o,
          idxv, rows, obuf, idx32, grows, sem):
        wid = lax.axis_index("s") * NC + lax.axis_index("c")

        def one_side(he, out):
            def chunk_body(c, _):
                base = wid * PPW + c * CH
                pltpu.sync_copy(he.at[pl.ds(base, CH)], idxv)

                def pair_body(i, _):
                    pltpu.async_copy(ent_t.at[idxv.at[i]], rows, sem).wait()

                    def acc_body(j, acc):
                        return tuple(
                            acc[v] + rows[j, pl.ds(v * 16, 16)]
                            for v in range(NV))

                    acc0 = tuple(jnp.zeros((16,), jnp.float32)
                                 for _ in range(NV))
                    acc = lax.fori_loop(0, K, acc_body, acc0)
                    for v in range(NV):
                        obuf[i, pl.ds(v * 16, 16)] = acc[v] * (1.0 / K)
                    return 0

                lax.fori_loop(0, CH, pair_body, 0)
                pltpu.sync_copy(obuf, out.at[pl.ds(base, CH)])
                return 0

            lax.fori_loop(0, PPW // CH, chunk_body, 0)

        def small_gather(src_idx, tab, out):
            base = wid * RPW
            pltpu.sync_copy(src_idx.at[pl.ds(base, RPW)], idx32)
            pltpu.async_copy(tab.at[idx32], grows, sem).wait()
            pltpu.sync_copy(grows, out.at[pl.ds(base, RPW)])

        one_side(he_s, neigh_s)
        one_side(he_o, neigh_o)
        small_gather(es_i, ent_t, es_o)
        small_gather(rs_i, rel_t, rs_o)
        small_gather(eo_i, ent_t, eo_o)
        small_gather(ro_i, rel_t, ro_o)

    return k


_sc_kernel = _sc_gather()


def _gru_body(neigh_ref, e_ref, r_ref, lens_ref, wet_ref, wnt_ref, wrt_ref,
              whh_ref, bih_ref, bhh_ref, out_ref):
    f32 = jnp.float32
    e = e_ref[...]
    r = r_ref[...]
    ge = (jnp.dot(e, wet_ref[...], preferred_element_type=f32)
          + jnp.dot(r, wrt_ref[...], preferred_element_type=f32)
          + bih_ref[...])
    lens = lens_ref[...]

    def step(t, h):
        nt = neigh_ref[t]
        gi = ge + jnp.dot(nt, wnt_ref[...], preferred_element_type=f32)
        gh = jnp.dot(h, whh_ref[...], preferred_element_type=f32) + bhh_ref[...]
        i_r, i_z, i_n = gi[:, :H], gi[:, H:2 * H], gi[:, 2 * H:]
        h_r, h_z, h_n = gh[:, :H], gh[:, H:2 * H], gh[:, 2 * H:]
        rg = jax.nn.sigmoid(i_r + h_r)
        z = jax.nn.sigmoid(i_z + h_z)
        n = jnp.tanh(i_n + rg * h_n)
        h_new = (1.0 - z) * n + z * h
        return jnp.where(lens > t, h_new, h)

    out_ref[...] = lax.fori_loop(0, T, step, jnp.zeros((B, H), f32))


_gru_call = pl.pallas_call(
    _gru_body,
    out_shape=jax.ShapeDtypeStruct((B, H), jnp.float32),
)


def _proj_body(x_ref, w_ref, lab_ref, pred_ref, loss_ref, m_ref, s_ref, g_ref):
    v = pl.program_id(0)
    f32 = jnp.float32

    @pl.when(v == 0)
    def _init():
        m_ref[...] = jnp.full((B, 1), -1e30, f32)
        s_ref[...] = jnp.zeros((B, 1), f32)
        g_ref[...] = jnp.zeros((B, 1), f32)

    logits = jnp.dot(x_ref[...], w_ref[...], preferred_element_type=f32)
    pred_ref[...] = logits

    bm = jnp.max(logits, axis=1, keepdims=True)
    m_old = m_ref[...]
    m_new = jnp.maximum(m_old, bm)
    s_ref[...] = (s_ref[...] * jnp.exp(m_old - m_new)
                  + jnp.sum(jnp.exp(logits - m_new), axis=1, keepdims=True))
    m_ref[...] = m_new

    cols = lax.broadcasted_iota(jnp.int32, (B, VB), 1) + v * VB
    match = cols == lab_ref[...]
    g_ref[...] = g_ref[...] + jnp.sum(jnp.where(match, logits, 0.0),
                                      axis=1, keepdims=True)

    @pl.when(v == pl.num_programs(0) - 1)
    def _fin():
        lse = m_ref[...] + jnp.log(s_ref[...])
        loss_ref[...] = (jnp.sum(lse - g_ref[...]) * (1.0 / B)).reshape(1, 1)


_proj_call = pl.pallas_call(
    _proj_body,
    grid=(VP // VB,),
    in_specs=[
        pl.BlockSpec((B, 608), lambda v: (0, 0)),
        pl.BlockSpec((608, VB), lambda v: (0, v)),
        pl.BlockSpec((B, 1), lambda v: (0, 0)),
    ],
    out_specs=[
        pl.BlockSpec((B, VB), lambda v: (0, v)),
        pl.BlockSpec((1, 1), lambda v: (0, 0)),
    ],
    out_shape=[
        jax.ShapeDtypeStruct((B, VP), jnp.float32),
        jax.ShapeDtypeStruct((1, 1), jnp.float32),
    ],
    scratch_shapes=[
        pltpu.VMEM((B, 1), jnp.float32),
        pltpu.VMEM((B, 1), jnp.float32),
        pltpu.VMEM((B, 1), jnp.float32),
    ],
)


def _prep_gru_weights(Wih, Whh, bih, bhh):
    z8 = jnp.zeros((8, 3 * H), jnp.float32)
    wet = jnp.concatenate([Wih[:, 0:H].T, z8], axis=0)
    wnt = jnp.concatenate([Wih[:, H:2 * H].T, z8], axis=0)
    wrt = jnp.concatenate([Wih[:, 2 * H:].T, z8], axis=0)
    whh = Whh.T
    return wet, wnt, wrt, whh, bih[None, :], bhh[None, :]


def _prep_proj_weights(Wlin, blin):
    wt = jnp.concatenate(
        [Wlin.T, jnp.zeros((600, VP - V), jnp.float32)], axis=1)
    brow = jnp.concatenate(
        [blin, jnp.full((VP - V,), -1e30, jnp.float32)])[None, :]
    zp = jnp.zeros((7, VP), jnp.float32)
    return jnp.concatenate([wt, brow, zp], axis=0)


def kernel(triplets, s_hist_ent, s_hist_len, o_hist_ent, o_hist_len,
           ent_embeds, rel_embeds, Wih_s, Whh_s, bih_s, bhh_s,
           Wih_o, Whh_o, bih_o, bhh_o, W_sub, b_sub, W_ob, b_ob):
    f32 = jnp.float32
    s_idx = jnp.argsort(-s_hist_len)
    o_idx = jnp.argsort(-o_hist_len)

    s = triplets[:, 0]
    r = triplets[:, 1]
    o = triplets[:, 2]

    he_s = s_hist_ent[s_idx].reshape(PAIRS, K)
    he_o = o_hist_ent[o_idx].reshape(PAIRS, K)
    lens_s = s_hist_len[s_idx].astype(jnp.int32)[:, None]
    lens_o = o_hist_len[o_idx].astype(jnp.int32)[:, None]
    ents_s, rels_s, targ_s = s[s_idx], r[s_idx], o[s_idx]
    ents_o, rels_o, targ_o = o[o_idx], r[o_idx], s[o_idx]

    zpad = jnp.zeros((V, HP - H), f32)
    ent_t = jnp.concatenate([ent_embeds, zpad], axis=1)
    rel_t = jnp.concatenate([rel_embeds, zpad], axis=1)

    neigh_s, neigh_o, e_s, r_s, e_o, r_o = _sc_kernel(
        he_s.astype(jnp.int32), he_o.astype(jnp.int32),
        ents_s.astype(jnp.int32), rels_s.astype(jnp.int32),
        ents_o.astype(jnp.int32), rels_o.astype(jnp.int32),
        ent_t, rel_t)

    neigh_s = neigh_s.reshape(B, T, HP).transpose(1, 0, 2)
    neigh_o = neigh_o.reshape(B, T, HP).transpose(1, 0, 2)

    h_s = _gru_call(neigh_s, e_s, r_s, lens_s,
                    *_prep_gru_weights(Wih_s, Whh_s, bih_s, bhh_s))
    h_o = _gru_call(neigh_o, e_o, r_o, lens_o,
                    *_prep_gru_weights(Wih_o, Whh_o, bih_o, bhh_o))

    ones = jnp.ones((B, 1), f32)
    z7 = jnp.zeros((B, 7), f32)
    x_s = jnp.concatenate([e_s[:, :H], h_s, r_s[:, :H], ones, z7], axis=1)
    x_o = jnp.concatenate([e_o[:, :H], h_o, r_o[:, :H], ones, z7], axis=1)

    ob_pred_p, loss_s = _proj_call(x_s, _prep_proj_weights(W_sub, b_sub),
                                   targ_s.astype(jnp.int32)[:, None])
    sub_pred_p, loss_o = _proj_call(x_o, _prep_proj_weights(W_ob, b_ob),
                                    targ_o.astype(jnp.int32)[:, None])

    loss = loss_s[0, 0] + loss_o[0, 0]
    return (loss, sub_pred_p[:, :V], ob_pred_p[:, :V], o_idx, s_idx)


# speedup vs baseline: 4.9265x; 1.4101x over previous
import functools
import jax
import jax.numpy as jnp
from jax import lax
from jax.experimental import pallas as pl
from jax.experimental.pallas import tpu as pltpu
from jax.experimental.pallas import tpu_sc as plsc

V = 10000
H = 200
HP = 208
T = 10
K = 64
B = 1024
NC, NS = 2, 16
NW = NC * NS
PAIRS = B * T
CH = 16
NCHUNK = PAIRS // CH
CPW = NCHUNK // NW
RPW = B // NW
NV = HP // 16

VB = 1024
VP = 10240


def _sc_gather():
    mesh = plsc.VectorSubcoreMesh(core_axis_name="c", subcore_axis_name="s")
    out_type = (
        jax.ShapeDtypeStruct((PAIRS, HP), jnp.float32),
        jax.ShapeDtypeStruct((PAIRS, HP), jnp.float32),
        jax.ShapeDtypeStruct((B, HP), jnp.float32),
        jax.ShapeDtypeStruct((B, HP), jnp.float32),
        jax.ShapeDtypeStruct((B, HP), jnp.float32),
        jax.ShapeDtypeStruct((B, HP), jnp.float32),
    )
    scratch = [
        pltpu.VMEM((CH, K), jnp.int32),
        pltpu.VMEM((2, K, HP), jnp.float32),
        pltpu.VMEM((CH, HP), jnp.float32),
        pltpu.VMEM((RPW,), jnp.int32),
        pltpu.VMEM((RPW, HP), jnp.float32),
        pltpu.SemaphoreType.DMA,
    ]

    @functools.partial(
        pl.kernel, mesh=mesh, out_type=out_type, scratch_types=scratch,
        compiler_params=pltpu.CompilerParams(use_tc_tiling_on_sc=False))
    def k(he_s, he_o, es_i, rs_i, eo_i, ro_i, ent_t, rel_t,
          neigh_s, neigh_o, es_o, rs_o, eo_o, ro_o,
          idxv, rows, obuf, idx32, grows, sem):
        wid = lax.axis_index("s") * NC + lax.axis_index("c")

        def one_side(he, out):
            def chunk_body(j, _):
                base = (wid + NW * j) * CH
                pltpu.sync_copy(he.at[pl.ds(base, CH)], idxv)
                pltpu.async_copy(ent_t.at[idxv.at[0]], rows.at[0], sem)

                def pair_body(i, _):
                    par = lax.rem(i, 2)

                    @pl.when(i + 1 < CH)
                    def _pre():
                        pltpu.async_copy(ent_t.at[idxv.at[i + 1]],
                                         rows.at[lax.rem(i + 1, 2)], sem)

                    pltpu.make_async_copy(ent_t.at[idxv.at[i]],
                                          rows.at[par], sem).wait()

                    def acc_body(j2, acc):
                        return tuple(
                            acc[v] + (rows[par, 2 * j2, pl.ds(v * 16, 16)]
                                      + rows[par, 2 * j2 + 1,
                                             pl.ds(v * 16, 16)])
                            for v in range(NV))

                    acc0 = tuple(jnp.zeros((16,), jnp.float32)
                                 for _ in range(NV))
                    acc = lax.fori_loop(0, K // 2, acc_body, acc0)
                    for v in range(NV):
                        obuf[i, pl.ds(v * 16, 16)] = acc[v] * (1.0 / K)
                    return 0

                lax.fori_loop(0, CH, pair_body, 0)
                pltpu.sync_copy(obuf, out.at[pl.ds(base, CH)])
                return 0

            lax.fori_loop(0, CPW, chunk_body, 0)

        def small_gather(src_idx, tab, out):
            base = wid * RPW
            pltpu.sync_copy(src_idx.at[pl.ds(base, RPW)], idx32)
            pltpu.async_copy(tab.at[idx32], grows, sem).wait()
            pltpu.sync_copy(grows, out.at[pl.ds(base, RPW)])

        one_side(he_s, neigh_s)
        one_side(he_o, neigh_o)
        small_gather(es_i, ent_t, es_o)
        small_gather(rs_i, rel_t, rs_o)
        small_gather(eo_i, ent_t, eo_o)
        small_gather(ro_i, rel_t, ro_o)

    return k


_sc_kernel = _sc_gather()


def _gru_body(neigh_ref, e_ref, r_ref, lens_ref, wet_ref, wnt_ref, wrt_ref,
              whh_ref, bih_ref, bhh_ref, out_ref):
    f32 = jnp.float32
    e = e_ref[...]
    r = r_ref[...]
    ge = (jnp.dot(e, wet_ref[...], preferred_element_type=f32)
          + jnp.dot(r, wrt_ref[...], preferred_element_type=f32)
          + bih_ref[...])
    lens = lens_ref[...]

    def step(t, h):
        nt = neigh_ref[t]
        gi = ge + jnp.dot(nt, wnt_ref[...], preferred_element_type=f32)
        gh = jnp.dot(h, whh_ref[...], preferred_element_type=f32) + bhh_ref[...]
        i_r, i_z, i_n = gi[:, :H], gi[:, H:2 * H], gi[:, 2 * H:]
        h_r, h_z, h_n = gh[:, :H], gh[:, H:2 * H], gh[:, 2 * H:]
        rg = jax.nn.sigmoid(i_r + h_r)
        z = jax.nn.sigmoid(i_z + h_z)
        n = jnp.tanh(i_n + rg * h_n)
        h_new = (1.0 - z) * n + z * h
        return jnp.where(lens > t, h_new, h)

    out_ref[...] = lax.fori_loop(0, T, step, jnp.zeros((B, H), f32))


_gru_call = pl.pallas_call(
    _gru_body,
    out_shape=jax.ShapeDtypeStruct((B, H), jnp.float32),
)


def _proj_body(x_ref, w_ref, lab_ref, pred_ref, loss_ref, m_ref, s_ref, g_ref):
    v = pl.program_id(0)
    f32 = jnp.float32

    @pl.when(v == 0)
    def _init():
        m_ref[...] = jnp.full((B, 1), -1e30, f32)
        s_ref[...] = jnp.zeros((B, 1), f32)
        g_ref[...] = jnp.zeros((B, 1), f32)

    logits = jnp.dot(x_ref[...], w_ref[...], preferred_element_type=f32)
    pred_ref[...] = logits

    bm = jnp.max(logits, axis=1, keepdims=True)
    m_old = m_ref[...]
    m_new = jnp.maximum(m_old, bm)
    s_ref[...] = (s_ref[...] * jnp.exp(m_old - m_new)
                  + jnp.sum(jnp.exp(logits - m_new), axis=1, keepdims=True))
    m_ref[...] = m_new

    cols = lax.broadcasted_iota(jnp.int32, (B, VB), 1) + v * VB
    match = cols == lab_ref[...]
    g_ref[...] = g_ref[...] + jnp.sum(jnp.where(match, logits, 0.0),
                                      axis=1, keepdims=True)

    @pl.when(v == pl.num_programs(0) - 1)
    def _fin():
        lse = m_ref[...] + jnp.log(s_ref[...])
        loss_ref[...] = (jnp.sum(lse - g_ref[...]) * (1.0 / B)).reshape(1, 1)


_proj_call = pl.pallas_call(
    _proj_body,
    grid=(VP // VB,),
    in_specs=[
        pl.BlockSpec((B, 608), lambda v: (0, 0)),
        pl.BlockSpec((608, VB), lambda v: (0, v)),
        pl.BlockSpec((B, 1), lambda v: (0, 0)),
    ],
    out_specs=[
        pl.BlockSpec((B, VB), lambda v: (0, v)),
        pl.BlockSpec((1, 1), lambda v: (0, 0)),
    ],
    out_shape=[
        jax.ShapeDtypeStruct((B, VP), jnp.float32),
        jax.ShapeDtypeStruct((1, 1), jnp.float32),
    ],
    scratch_shapes=[
        pltpu.VMEM((B, 1), jnp.float32),
        pltpu.VMEM((B, 1), jnp.float32),
        pltpu.VMEM((B, 1), jnp.float32),
    ],
)


def _prep_gru_weights(Wih, Whh, bih, bhh):
    z8 = jnp.zeros((8, 3 * H), jnp.float32)
    wet = jnp.concatenate([Wih[:, 0:H].T, z8], axis=0)
    wnt = jnp.concatenate([Wih[:, H:2 * H].T, z8], axis=0)
    wrt = jnp.concatenate([Wih[:, 2 * H:].T, z8], axis=0)
    whh = Whh.T
    return wet, wnt, wrt, whh, bih[None, :], bhh[None, :]


def _prep_proj_weights(Wlin, blin):
    wt = jnp.concatenate(
        [Wlin.T, jnp.zeros((600, VP - V), jnp.float32)], axis=1)
    brow = jnp.concatenate(
        [blin, jnp.full((VP - V,), -1e30, jnp.float32)])[None, :]
    zp = jnp.zeros((7, VP), jnp.float32)
    return jnp.concatenate([wt, brow, zp], axis=0)


def kernel(triplets, s_hist_ent, s_hist_len, o_hist_ent, o_hist_len,
           ent_embeds, rel_embeds, Wih_s, Whh_s, bih_s, bhh_s,
           Wih_o, Whh_o, bih_o, bhh_o, W_sub, b_sub, W_ob, b_ob):
    f32 = jnp.float32
    s_idx = jnp.argsort(-s_hist_len)
    o_idx = jnp.argsort(-o_hist_len)

    s = triplets[:, 0]
    r = triplets[:, 1]
    o = triplets[:, 2]

    he_s = s_hist_ent[s_idx].transpose(1, 0, 2).reshape(PAIRS, K)
    he_o = o_hist_ent[o_idx].transpose(1, 0, 2).reshape(PAIRS, K)
    lens_s = s_hist_len[s_idx].astype(jnp.int32)[:, None]
    lens_o = o_hist_len[o_idx].astype(jnp.int32)[:, None]
    ents_s, rels_s, targ_s = s[s_idx], r[s_idx], o[s_idx]
    ents_o, rels_o, targ_o = o[o_idx], r[o_idx], s[o_idx]

    zpad = jnp.zeros((V, HP - H), f32)
    ent_t = jnp.concatenate([ent_embeds, zpad], axis=1)
    rel_t = jnp.concatenate([rel_embeds, zpad], axis=1)

    neigh_s, neigh_o, e_s, r_s, e_o, r_o = _sc_kernel(
        he_s.astype(jnp.int32), he_o.astype(jnp.int32),
        ents_s.astype(jnp.int32), rels_s.astype(jnp.int32),
        ents_o.astype(jnp.int32), rels_o.astype(jnp.int32),
        ent_t, rel_t)

    neigh_s = neigh_s.reshape(T, B, HP)
    neigh_o = neigh_o.reshape(T, B, HP)

    h_s = _gru_call(neigh_s, e_s, r_s, lens_s,
                    *_prep_gru_weights(Wih_s, Whh_s, bih_s, bhh_s))
    h_o = _gru_call(neigh_o, e_o, r_o, lens_o,
                    *_prep_gru_weights(Wih_o, Whh_o, bih_o, bhh_o))

    ones = jnp.ones((B, 1), f32)
    z7 = jnp.zeros((B, 7), f32)
    x_s = jnp.concatenate([e_s[:, :H], h_s, r_s[:, :H], ones, z7], axis=1)
    x_o = jnp.concatenate([e_o[:, :H], h_o, r_o[:, :H], ones, z7], axis=1)

    ob_pred_p, loss_s = _proj_call(x_s, _prep_proj_weights(W_sub, b_sub),
                                   targ_s.astype(jnp.int32)[:, None])
    sub_pred_p, loss_o = _proj_call(x_o, _prep_proj_weights(W_ob, b_ob),
                                    targ_o.astype(jnp.int32)[:, None])

    loss = loss_s[0, 0] + loss_o[0, 0]
    return (loss, sub_pred_p[:, :V], ob_pred_p[:, :V], o_idx, s_idx)


# speedup vs baseline: 5.4881x; 1.1140x over previous
import functools
import jax
import jax.numpy as jnp
from jax import lax
from jax.experimental import pallas as pl
from jax.experimental.pallas import tpu as pltpu
from jax.experimental.pallas import tpu_sc as plsc

V = 10000
H = 200
HP = 208
T = 10
K = 64
B = 1024
NC, NS = 2, 16
NW = NC * NS
PAIRS = B * T
CH = 16
NCHUNK = PAIRS // CH
CPW = NCHUNK // NW
RPW = B // NW
NV = HP // 16

VB = 1024
VP = 10240


def _sc_gather():
    mesh = plsc.VectorSubcoreMesh(core_axis_name="c", subcore_axis_name="s")
    out_type = (
        jax.ShapeDtypeStruct((PAIRS, HP), jnp.float32),
        jax.ShapeDtypeStruct((B, HP), jnp.float32),
        jax.ShapeDtypeStruct((B, HP), jnp.float32),
    )
    scratch = [
        pltpu.VMEM((CH, K), jnp.int32),
        pltpu.VMEM((2, K, HP), jnp.float32),
        pltpu.VMEM((CH, HP), jnp.float32),
        pltpu.VMEM((RPW,), jnp.int32),
        pltpu.VMEM((RPW, HP), jnp.float32),
        pltpu.SemaphoreType.DMA,
    ]

    @functools.partial(
        pl.kernel, mesh=mesh, out_type=out_type, scratch_types=scratch,
        compiler_params=pltpu.CompilerParams(use_tc_tiling_on_sc=False))
    def k(he, e_i, r_i, ent_t, rel_t, neigh, e_o, r_o,
          idxv, rows, obuf, idx32, grows, sem):
        wid = lax.axis_index("s") * NC + lax.axis_index("c")

        def one_side(he, out):
            def chunk_body(j, _):
                base = (wid + NW * j) * CH
                pltpu.sync_copy(he.at[pl.ds(base, CH)], idxv)
                pltpu.async_copy(ent_t.at[idxv.at[0]], rows.at[0], sem)

                def pair_body(i, _):
                    par = lax.rem(i, 2)

                    @pl.when(i + 1 < CH)
                    def _pre():
                        pltpu.async_copy(ent_t.at[idxv.at[i + 1]],
                                         rows.at[lax.rem(i + 1, 2)], sem)

                    pltpu.make_async_copy(ent_t.at[idxv.at[i]],
                                          rows.at[par], sem).wait()

                    def acc_body(j2, acc):
                        return tuple(
                            acc[v] + (rows[par, 2 * j2, pl.ds(v * 16, 16)]
                                      + rows[par, 2 * j2 + 1,
                                             pl.ds(v * 16, 16)])
                            for v in range(NV))

                    acc0 = tuple(jnp.zeros((16,), jnp.float32)
                                 for _ in range(NV))
                    acc = lax.fori_loop(0, K // 2, acc_body, acc0)
                    for v in range(NV):
                        obuf[i, pl.ds(v * 16, 16)] = acc[v] * (1.0 / K)
                    return 0

                lax.fori_loop(0, CH, pair_body, 0)
                pltpu.sync_copy(obuf, out.at[pl.ds(base, CH)])
                return 0

            lax.fori_loop(0, CPW, chunk_body, 0)

        def small_gather(src_idx, tab, out):
            base = wid * RPW
            pltpu.sync_copy(src_idx.at[pl.ds(base, RPW)], idx32)
            pltpu.async_copy(tab.at[idx32], grows, sem).wait()
            pltpu.sync_copy(grows, out.at[pl.ds(base, RPW)])

        one_side(he, neigh)
        small_gather(e_i, ent_t, e_o)
        small_gather(r_i, rel_t, r_o)

    return k


_sc_kernel = _sc_gather()


def _gru_body(neigh_ref, e_ref, r_ref, lens_ref, wet_ref, wnt_ref, wrt_ref,
              whh_ref, bih_ref, bhh_ref, out_ref):
    f32 = jnp.float32
    e = e_ref[...]
    r = r_ref[...]
    ge = (jnp.dot(e, wet_ref[...], preferred_element_type=f32)
          + jnp.dot(r, wrt_ref[...], preferred_element_type=f32)
          + bih_ref[...])
    lens = lens_ref[...]

    def step(t, h):
        nt = neigh_ref[t]
        gi = ge + jnp.dot(nt, wnt_ref[...], preferred_element_type=f32)
        gh = jnp.dot(h, whh_ref[...], preferred_element_type=f32) + bhh_ref[...]
        i_r, i_z, i_n = gi[:, :H], gi[:, H:2 * H], gi[:, 2 * H:]
        h_r, h_z, h_n = gh[:, :H], gh[:, H:2 * H], gh[:, 2 * H:]
        rg = jax.nn.sigmoid(i_r + h_r)
        z = jax.nn.sigmoid(i_z + h_z)
        n = jnp.tanh(i_n + rg * h_n)
        h_new = (1.0 - z) * n + z * h
        return jnp.where(lens > t, h_new, h)

    out_ref[...] = lax.fori_loop(0, T, step, jnp.zeros((B, H), f32))


_gru_call = pl.pallas_call(
    _gru_body,
    out_shape=jax.ShapeDtypeStruct((B, H), jnp.float32),
)


def _proj_body(x_ref, w_ref, lab_ref, pred_ref, loss_ref, m_ref, s_ref, g_ref):
    v = pl.program_id(0)
    f32 = jnp.float32

    @pl.when(v == 0)
    def _init():
        m_ref[...] = jnp.full((B, 1), -1e30, f32)
        s_ref[...] = jnp.zeros((B, 1), f32)
        g_ref[...] = jnp.zeros((B, 1), f32)

    logits = jnp.dot(x_ref[...], w_ref[...], preferred_element_type=f32)
    pred_ref[...] = logits

    bm = jnp.max(logits, axis=1, keepdims=True)
    m_old = m_ref[...]
    m_new = jnp.maximum(m_old, bm)
    s_ref[...] = (s_ref[...] * jnp.exp(m_old - m_new)
                  + jnp.sum(jnp.exp(logits - m_new), axis=1, keepdims=True))
    m_ref[...] = m_new

    cols = lax.broadcasted_iota(jnp.int32, (B, VB), 1) + v * VB
    match = cols == lab_ref[...]
    g_ref[...] = g_ref[...] + jnp.sum(jnp.where(match, logits, 0.0),
                                      axis=1, keepdims=True)

    @pl.when(v == pl.num_programs(0) - 1)
    def _fin():
        lse = m_ref[...] + jnp.log(s_ref[...])
        loss_ref[...] = (jnp.sum(lse - g_ref[...]) * (1.0 / B)).reshape(1, 1)


_proj_call = pl.pallas_call(
    _proj_body,
    grid=(VP // VB,),
    in_specs=[
        pl.BlockSpec((B, 608), lambda v: (0, 0)),
        pl.BlockSpec((608, VB), lambda v: (0, v)),
        pl.BlockSpec((B, 1), lambda v: (0, 0)),
    ],
    out_specs=[
        pl.BlockSpec((B, VB), lambda v: (0, v)),
        pl.BlockSpec((1, 1), lambda v: (0, 0)),
    ],
    out_shape=[
        jax.ShapeDtypeStruct((B, VP), jnp.float32),
        jax.ShapeDtypeStruct((1, 1), jnp.float32),
    ],
    scratch_shapes=[
        pltpu.VMEM((B, 1), jnp.float32),
        pltpu.VMEM((B, 1), jnp.float32),
        pltpu.VMEM((B, 1), jnp.float32),
    ],
)


def _prep_gru_weights(Wih, Whh, bih, bhh):
    z8 = jnp.zeros((8, 3 * H), jnp.float32)
    wet = jnp.concatenate([Wih[:, 0:H].T, z8], axis=0)
    wnt = jnp.concatenate([Wih[:, H:2 * H].T, z8], axis=0)
    wrt = jnp.concatenate([Wih[:, 2 * H:].T, z8], axis=0)
    whh = Whh.T
    return wet, wnt, wrt, whh, bih[None, :], bhh[None, :]


def _prep_proj_weights(Wlin, blin):
    wt = jnp.concatenate(
        [Wlin.T, jnp.zeros((600, VP - V), jnp.float32)], axis=1)
    brow = jnp.concatenate(
        [blin, jnp.full((VP - V,), -1e30, jnp.float32)])[None, :]
    zp = jnp.zeros((7, VP), jnp.float32)
    return jnp.concatenate([wt, brow, zp], axis=0)


def kernel(triplets, s_hist_ent, s_hist_len, o_hist_ent, o_hist_len,
           ent_embeds, rel_embeds, Wih_s, Whh_s, bih_s, bhh_s,
           Wih_o, Whh_o, bih_o, bhh_o, W_sub, b_sub, W_ob, b_ob):
    f32 = jnp.float32
    s_idx = jnp.argsort(-s_hist_len)
    o_idx = jnp.argsort(-o_hist_len)

    s = triplets[:, 0]
    r = triplets[:, 1]
    o = triplets[:, 2]

    he_s = s_hist_ent[s_idx].transpose(1, 0, 2).reshape(PAIRS, K)
    he_o = o_hist_ent[o_idx].transpose(1, 0, 2).reshape(PAIRS, K)
    lens_s = s_hist_len[s_idx].astype(jnp.int32)[:, None]
    lens_o = o_hist_len[o_idx].astype(jnp.int32)[:, None]
    ents_s, rels_s, targ_s = s[s_idx], r[s_idx], o[s_idx]
    ents_o, rels_o, targ_o = o[o_idx], r[o_idx], s[o_idx]

    zpad = jnp.zeros((V, HP - H), f32)
    ent_t = jnp.concatenate([ent_embeds, zpad], axis=1)
    rel_t = jnp.concatenate([rel_embeds, zpad], axis=1)

    neigh_s, e_s, r_s = _sc_kernel(
        he_s.astype(jnp.int32), ents_s.astype(jnp.int32),
        rels_s.astype(jnp.int32), ent_t, rel_t)
    neigh_o, e_o, r_o = _sc_kernel(
        he_o.astype(jnp.int32), ents_o.astype(jnp.int32),
        rels_o.astype(jnp.int32), ent_t, rel_t)

    neigh_s = neigh_s.reshape(T, B, HP)
    neigh_o = neigh_o.reshape(T, B, HP)

    h_s = _gru_call(neigh_s, e_s, r_s, lens_s,
                    *_prep_gru_weights(Wih_s, Whh_s, bih_s, bhh_s))
    h_o = _gru_call(neigh_o, e_o, r_o, lens_o,
                    *_prep_gru_weights(Wih_o, Whh_o, bih_o, bhh_o))

    ones = jnp.ones((B, 1), f32)
    z7 = jnp.zeros((B, 7), f32)
    x_s = jnp.concatenate([e_s[:, :H], h_s, r_s[:, :H], ones, z7], axis=1)
    x_o = jnp.concatenate([e_o[:, :H], h_o, r_o[:, :H], ones, z7], axis=1)

    ob_pred_p, loss_s = _proj_call(x_s, _prep_proj_weights(W_sub, b_sub),
                                   targ_s.astype(jnp.int32)[:, None])
    sub_pred_p, loss_o = _proj_call(x_o, _prep_proj_weights(W_ob, b_ob),
                                    targ_o.astype(jnp.int32)[:, None])

    loss = loss_s[0, 0] + loss_o[0, 0]
    return (loss, sub_pred_p[:, :V], ob_pred_p[:, :V], o_idx, s_idx)


# speedup vs baseline: 5.5870x; 1.0180x over previous
import functools
import jax
import jax.numpy as jnp
from jax import lax
from jax.experimental import pallas as pl
from jax.experimental.pallas import tpu as pltpu
from jax.experimental.pallas import tpu_sc as plsc

V = 10000
H = 200
HP = 208
T = 10
K = 64
B = 1024
NC, NS = 2, 16
NW = NC * NS
PAIRS = B * T
CH = 32
NCHUNK = PAIRS // CH
CPW = NCHUNK // NW
RPW = B // NW
NV = HP // 16

VB = 1024
VP = 10240


def _sc_gather():
    mesh = plsc.VectorSubcoreMesh(core_axis_name="c", subcore_axis_name="s")
    out_type = (
        jax.ShapeDtypeStruct((PAIRS, HP), jnp.float32),
        jax.ShapeDtypeStruct((B, HP), jnp.float32),
        jax.ShapeDtypeStruct((B, HP), jnp.float32),
    )
    scratch = [
        pltpu.VMEM((CH, K), jnp.int32),
        pltpu.VMEM((2, K, HP), jnp.float32),
        pltpu.VMEM((CH, HP), jnp.float32),
        pltpu.VMEM((RPW,), jnp.int32),
        pltpu.VMEM((RPW, HP), jnp.float32),
        pltpu.SemaphoreType.DMA,
    ]

    @functools.partial(
        pl.kernel, mesh=mesh, out_type=out_type, scratch_types=scratch,
        compiler_params=pltpu.CompilerParams(use_tc_tiling_on_sc=False))
    def k(he, e_i, r_i, ent_t, rel_t, neigh, e_o, r_o,
          idxv, rows, obuf, idx32, grows, sem):
        wid = lax.axis_index("s") * NC + lax.axis_index("c")

        def one_side(he, out):
            def chunk_body(j, _):
                base = (wid + NW * j) * CH
                pltpu.sync_copy(he.at[pl.ds(base, CH)], idxv)
                pltpu.async_copy(ent_t.at[idxv.at[0]], rows.at[0], sem)

                def pair_body(i, _):
                    par = lax.rem(i, 2)

                    @pl.when(i + 1 < CH)
                    def _pre():
                        pltpu.async_copy(ent_t.at[idxv.at[i + 1]],
                                         rows.at[lax.rem(i + 1, 2)], sem)

                    pltpu.make_async_copy(ent_t.at[idxv.at[i]],
                                          rows.at[par], sem).wait()

                    def acc_body(j2, acc):
                        return tuple(
                            acc[v]
                            + ((rows[par, 4 * j2, pl.ds(v * 16, 16)]
                                + rows[par, 4 * j2 + 1, pl.ds(v * 16, 16)])
                               + (rows[par, 4 * j2 + 2, pl.ds(v * 16, 16)]
                                  + rows[par, 4 * j2 + 3, pl.ds(v * 16, 16)]))
                            for v in range(NV))

                    acc0 = tuple(jnp.zeros((16,), jnp.float32)
                                 for _ in range(NV))
                    acc = lax.fori_loop(0, K // 4, acc_body, acc0)
                    for v in range(NV):
                        obuf[i, pl.ds(v * 16, 16)] = acc[v] * (1.0 / K)
                    return 0

                lax.fori_loop(0, CH, pair_body, 0)
                pltpu.sync_copy(obuf, out.at[pl.ds(base, CH)])
                return 0

            lax.fori_loop(0, CPW, chunk_body, 0)

        def small_gather(src_idx, tab, out):
            base = wid * RPW
            pltpu.sync_copy(src_idx.at[pl.ds(base, RPW)], idx32)
            pltpu.async_copy(tab.at[idx32], grows, sem).wait()
            pltpu.sync_copy(grows, out.at[pl.ds(base, RPW)])

        one_side(he, neigh)
        small_gather(e_i, ent_t, e_o)
        small_gather(r_i, rel_t, r_o)

    return k


_sc_kernel = _sc_gather()


def _gru_body(neigh_ref, e_ref, r_ref, lens_ref, wet_ref, wnt_ref, wrt_ref,
              whh_ref, bih_ref, bhh_ref, out_ref):
    f32 = jnp.float32
    e = e_ref[...]
    r = r_ref[...]
    ge = (jnp.dot(e, wet_ref[...], preferred_element_type=f32)
          + jnp.dot(r, wrt_ref[...], preferred_element_type=f32)
          + bih_ref[...])
    lens = lens_ref[...]

    def step(t, h):
        nt = neigh_ref[t]
        gi = ge + jnp.dot(nt, wnt_ref[...], preferred_element_type=f32)
        gh = jnp.dot(h, whh_ref[...], preferred_element_type=f32) + bhh_ref[...]
        i_r, i_z, i_n = gi[:, :H], gi[:, H:2 * H], gi[:, 2 * H:]
        h_r, h_z, h_n = gh[:, :H], gh[:, H:2 * H], gh[:, 2 * H:]
        rg = jax.nn.sigmoid(i_r + h_r)
        z = jax.nn.sigmoid(i_z + h_z)
        n = jnp.tanh(i_n + rg * h_n)
        h_new = (1.0 - z) * n + z * h
        return jnp.where(lens > t, h_new, h)

    out_ref[...] = lax.fori_loop(0, T, step, jnp.zeros((B, H), f32))


_gru_call = pl.pallas_call(
    _gru_body,
    out_shape=jax.ShapeDtypeStruct((B, H), jnp.float32),
)


def _proj_body(x_ref, w_ref, lab_ref, pred_ref, loss_ref, m_ref, s_ref, g_ref):
    v = pl.program_id(0)
    f32 = jnp.float32

    @pl.when(v == 0)
    def _init():
        m_ref[...] = jnp.full((B, 1), -1e30, f32)
        s_ref[...] = jnp.zeros((B, 1), f32)
        g_ref[...] = jnp.zeros((B, 1), f32)

    logits = jnp.dot(x_ref[...], w_ref[...], preferred_element_type=f32)
    pred_ref[...] = logits

    bm = jnp.max(logits, axis=1, keepdims=True)
    m_old = m_ref[...]
    m_new = jnp.maximum(m_old, bm)
    s_ref[...] = (s_ref[...] * jnp.exp(m_old - m_new)
                  + jnp.sum(jnp.exp(logits - m_new), axis=1, keepdims=True))
    m_ref[...] = m_new

    cols = lax.broadcasted_iota(jnp.int32, (B, VB), 1) + v * VB
    match = cols == lab_ref[...]
    g_ref[...] = g_ref[...] + jnp.sum(jnp.where(match, logits, 0.0),
                                      axis=1, keepdims=True)

    @pl.when(v == pl.num_programs(0) - 1)
    def _fin():
        lse = m_ref[...] + jnp.log(s_ref[...])
        loss_ref[...] = (jnp.sum(lse - g_ref[...]) * (1.0 / B)).reshape(1, 1)


_proj_call = pl.pallas_call(
    _proj_body,
    grid=(VP // VB,),
    in_specs=[
        pl.BlockSpec((B, 608), lambda v: (0, 0)),
        pl.BlockSpec((608, VB), lambda v: (0, v)),
        pl.BlockSpec((B, 1), lambda v: (0, 0)),
    ],
    out_specs=[
        pl.BlockSpec((B, VB), lambda v: (0, v)),
        pl.BlockSpec((1, 1), lambda v: (0, 0)),
    ],
    out_shape=[
        jax.ShapeDtypeStruct((B, VP), jnp.float32),
        jax.ShapeDtypeStruct((1, 1), jnp.float32),
    ],
    scratch_shapes=[
        pltpu.VMEM((B, 1), jnp.float32),
        pltpu.VMEM((B, 1), jnp.float32),
        pltpu.VMEM((B, 1), jnp.float32),
    ],
)


def _prep_gru_weights(Wih, Whh, bih, bhh):
    z8 = jnp.zeros((8, 3 * H), jnp.float32)
    wet = jnp.concatenate([Wih[:, 0:H].T, z8], axis=0)
    wnt = jnp.concatenate([Wih[:, H:2 * H].T, z8], axis=0)
    wrt = jnp.concatenate([Wih[:, 2 * H:].T, z8], axis=0)
    whh = Whh.T
    return wet, wnt, wrt, whh, bih[None, :], bhh[None, :]


def _prep_proj_weights(Wlin, blin):
    wt = jnp.concatenate(
        [Wlin.T, jnp.zeros((600, VP - V), jnp.float32)], axis=1)
    brow = jnp.concatenate(
        [blin, jnp.full((VP - V,), -1e30, jnp.float32)])[None, :]
    zp = jnp.zeros((7, VP), jnp.float32)
    return jnp.concatenate([wt, brow, zp], axis=0)


def kernel(triplets, s_hist_ent, s_hist_len, o_hist_ent, o_hist_len,
           ent_embeds, rel_embeds, Wih_s, Whh_s, bih_s, bhh_s,
           Wih_o, Whh_o, bih_o, bhh_o, W_sub, b_sub, W_ob, b_ob):
    f32 = jnp.float32
    s_idx = jnp.argsort(-s_hist_len)
    o_idx = jnp.argsort(-o_hist_len)

    s = triplets[:, 0]
    r = triplets[:, 1]
    o = triplets[:, 2]

    he_s = s_hist_ent[s_idx].transpose(1, 0, 2).reshape(PAIRS, K)
    he_o = o_hist_ent[o_idx].transpose(1, 0, 2).reshape(PAIRS, K)
    lens_s = s_hist_len[s_idx].astype(jnp.int32)[:, None]
    lens_o = o_hist_len[o_idx].astype(jnp.int32)[:, None]
    ents_s, rels_s, targ_s = s[s_idx], r[s_idx], o[s_idx]
    ents_o, rels_o, targ_o = o[o_idx], r[o_idx], s[o_idx]

    zpad = jnp.zeros((V, HP - H), f32)
    ent_t = jnp.concatenate([ent_embeds, zpad], axis=1)
    rel_t = jnp.concatenate([rel_embeds, zpad], axis=1)

    neigh_s, e_s, r_s = _sc_kernel(
        he_s.astype(jnp.int32), ents_s.astype(jnp.int32),
        rels_s.astype(jnp.int32), ent_t, rel_t)
    neigh_o, e_o, r_o = _sc_kernel(
        he_o.astype(jnp.int32), ents_o.astype(jnp.int32),
        rels_o.astype(jnp.int32), ent_t, rel_t)

    neigh_s = neigh_s.reshape(T, B, HP)
    neigh_o = neigh_o.reshape(T, B, HP)

    h_s = _gru_call(neigh_s, e_s, r_s, lens_s,
                    *_prep_gru_weights(Wih_s, Whh_s, bih_s, bhh_s))
    h_o = _gru_call(neigh_o, e_o, r_o, lens_o,
                    *_prep_gru_weights(Wih_o, Whh_o, bih_o, bhh_o))

    ones = jnp.ones((B, 1), f32)
    z7 = jnp.zeros((B, 7), f32)
    x_s = jnp.concatenate([e_s[:, :H], h_s, r_s[:, :H], ones, z7], axis=1)
    x_o = jnp.concatenate([e_o[:, :H], h_o, r_o[:, :H], ones, z7], axis=1)

    ob_pred_p, loss_s = _proj_call(x_s, _prep_proj_weights(W_sub, b_sub),
                                   targ_s.astype(jnp.int32)[:, None])
    sub_pred_p, loss_o = _proj_call(x_o, _prep_proj_weights(W_ob, b_ob),
                                    targ_o.astype(jnp.int32)[:, None])

    loss = loss_s[0, 0] + loss_o[0, 0]
    return (loss, sub_pred_p[:, :V], ob_pred_p[:, :V], o_idx, s_idx)
